# baseline (speedup 1.0000x reference)
import jax
import jax.numpy as jnp
from jax import lax
from jax.experimental import pallas as pl
from jax.experimental.pallas import tpu as pltpu
from jax.experimental.pallas import tpu_sc as plsc

VOCAB = 1000000
DIM = 64
B = 4096
L = 50
LP = 56

NC, NS = 2, 16
NW = NC * NS
ROWS_W = B // NW
XR = 16
SUP = XR * L
NSUP = ROWS_W // XR
NG = SUP // 16


def _body(x_hbm, table_hbm, iv_hbm, out_hbm,
          ixf0, ixf1, iv_v, idxf0, idxf1, rbuf, isem, gsem, osem):
    wid = lax.axis_index("s") * NC + lax.axis_index("c")
    row0 = wid * ROWS_W
    ixf = (ixf0, ixf1)
    idxf = (idxf0, idxf1)
    pltpu.sync_copy(iv_hbm, iv_v)

    def stage(s, b):
        return [pltpu.async_copy(
            x_hbm.at[row0 + s * XR + j], ixf[b].at[pl.ds(j * LP, LP)],
            isem.at[b]) for j in range(XR)]

    def repack(b):
        for g in range(NG):
            iv = iv_v[pl.ds(g * 16, 16)]
            vals = plsc.load_gather(ixf[b], [iv])
            idxf[b][pl.ds(g * 16, 16)] = vals

    def gather(b):
        return pltpu.async_copy(
            table_hbm.at[idxf[b]], rbuf.at[b], gsem.at[b])

    def writeback(s, b):
        return [pltpu.async_copy(
            rbuf.at[b, pl.ds(j * L, L)],
            out_hbm.at[row0 + s * XR + j], osem.at[b]) for j in range(XR)]

    for d in stage(0, 0):
        d.wait()
    repack(0)
    gd = {0: gather(0)}
    id_ = {}
    wd = {}
    for s in range(NSUP):
        b = s % 2
        nb = (s + 1) % 2
        if s + 1 < NSUP:
            id_[s + 1] = stage(s + 1, nb)
        gd[s].wait()
        if s + 1 < NSUP:
            for d in id_[s + 1]:
                d.wait()
            repack(nb)
            if s - 1 >= 0:
                for d in wd[s - 1]:
                    d.wait()
            gd[s + 1] = gather(nb)
        wd[s] = writeback(s, b)
    for d in wd[NSUP - 2]:
        d.wait()
    for d in wd[NSUP - 1]:
        d.wait()


@jax.jit
def _embed(x, E):
    mesh = plsc.VectorSubcoreMesh(core_axis_name="c", subcore_axis_name="s")
    iv_tab = jnp.array(
        [(p // L) * LP + p % L for p in range(SUP)], jnp.int32)
    return pl.kernel(
        _body,
        out_type=jax.ShapeDtypeStruct((B, L, DIM), jnp.float32),
        mesh=mesh,
        scratch_types=[
            pltpu.VMEM((XR * LP,), jnp.int32),
            pltpu.VMEM((XR * LP,), jnp.int32),
            pltpu.VMEM((SUP,), jnp.int32),
            pltpu.VMEM((SUP,), jnp.int32),
            pltpu.VMEM((SUP,), jnp.int32),
            pltpu.VMEM((2, SUP, DIM), jnp.float32),
            pltpu.SemaphoreType.DMA((2,)),
            pltpu.SemaphoreType.DMA((2,)),
            pltpu.SemaphoreType.DMA((2,)),
        ],
        compiler_params=pltpu.CompilerParams(
            use_tc_tiling_on_sc=False, needs_layout_passes=False),
    )(x, E, iv_tab)


def kernel(x, E):
    xp = jnp.pad(x.astype(jnp.int32), ((0, 0), (0, LP - L)))
    return _embed(xp, E)


# speedup vs baseline: 1.0122x; 1.0122x over previous
import functools

import jax
import jax.numpy as jnp
from jax import lax
from jax.experimental import pallas as pl
from jax.experimental.pallas import tpu as pltpu
from jax.experimental.pallas import tpu_sc as plsc

VOCAB = 1000000
DIM = 64
B = 4096
L = 50

NC, NS = 2, 16
NW = NC * NS
TOTAL = B * L
PER_W = TOTAL // NW
N_CHUNK = 8
CH = PER_W // N_CHUNK


def _body(idx_hbm, table_hbm, out_hbm, idx_v, bufs, gsem, wsem):
    wid = lax.axis_index("s") * NC + lax.axis_index("c")
    base = wid * PER_W
    pltpu.sync_copy(idx_hbm.at[pl.ds(base, PER_W)], idx_v)

    gd = {}
    wd = {}
    for c in range(N_CHUNK):
        b = c % 2
        if c == 0:
            gd[0] = pltpu.async_copy(
                table_hbm.at[idx_v.at[pl.ds(0, CH)]], bufs.at[0], gsem.at[0])
        if c + 1 < N_CHUNK:
            nb = (c + 1) % 2
            if c - 1 >= 0:
                wd[c - 1].wait()
            gd[c + 1] = pltpu.async_copy(
                table_hbm.at[idx_v.at[pl.ds((c + 1) * CH, CH)]],
                bufs.at[nb], gsem.at[nb])
        gd[c].wait()
        wd[c] = pltpu.async_copy(
            bufs.at[b], out_hbm.at[pl.ds(base + c * CH, CH)], wsem.at[b])
    wd[N_CHUNK - 2].wait()
    wd[N_CHUNK - 1].wait()


@jax.jit
def _embed(x_flat, E):
    mesh = plsc.VectorSubcoreMesh(core_axis_name="c", subcore_axis_name="s")
    return pl.kernel(
        _body,
        out_type=jax.ShapeDtypeStruct((TOTAL, DIM), jnp.float32),
        mesh=mesh,
        scratch_types=[
            pltpu.VMEM((PER_W,), jnp.int32),
            pltpu.VMEM((2, CH, DIM), jnp.float32),
            pltpu.SemaphoreType.DMA((2,)),
            pltpu.SemaphoreType.DMA((2,)),
        ],
        compiler_params=pltpu.CompilerParams(use_tc_tiling_on_sc=False),
    )(x_flat, E)


def kernel(x, E):
    flat = x.reshape(-1).astype(jnp.int32)
    out = _embed(flat, E)
    return out.reshape(x.shape[0], x.shape[1], E.shape[1])


# speedup vs baseline: 1.0126x; 1.0004x over previous
import jax
import jax.numpy as jnp
from jax import lax
from jax.experimental import pallas as pl
from jax.experimental.pallas import tpu as pltpu
from jax.experimental.pallas import tpu_sc as plsc

VOCAB = 1000000
DIM = 64
B = 4096
L = 50

NC, NS = 2, 16
NW = NC * NS
TOTAL = B * L
PER_W = TOTAL // NW
N_CHUNK = 8
CH = PER_W // N_CHUNK


def _body(idx_hbm, table_hbm, out_hbm, idx_v, bufs, gsem, wsem):
    wid = lax.axis_index("s") * NC + lax.axis_index("c")
    base = wid * PER_W
    pltpu.sync_copy(idx_hbm.at[pl.ds(base, PER_W)], idx_v)

    gd = {}
    wd = {}
    for c in range(N_CHUNK):
        b = c % 2
        if c == 0:
            gd[0] = pltpu.async_copy(
                table_hbm.at[idx_v.at[pl.ds(0, CH)]], bufs.at[0], gsem.at[0])
        if c + 1 < N_CHUNK:
            nb = (c + 1) % 2
            if c - 1 >= 0:
                wd[c - 1].wait()
            gd[c + 1] = pltpu.async_copy(
                table_hbm.at[idx_v.at[pl.ds((c + 1) * CH, CH)]],
                bufs.at[nb], gsem.at[nb])
        gd[c].wait()
        wd[c] = pltpu.async_copy(
            bufs.at[b], out_hbm.at[pl.ds(base + c * CH, CH)], wsem.at[b])
    wd[N_CHUNK - 2].wait()
    wd[N_CHUNK - 1].wait()


@jax.jit
def _embed(x_flat, E):
    mesh = plsc.VectorSubcoreMesh(core_axis_name="c", subcore_axis_name="s")
    return pl.kernel(
        _body,
        out_type=jax.ShapeDtypeStruct((TOTAL, DIM), jnp.float32),
        mesh=mesh,
        scratch_types=[
            pltpu.VMEM((PER_W,), jnp.int32),
            pltpu.VMEM((2, CH, DIM), jnp.float32),
            pltpu.SemaphoreType.DMA((2,)),
            pltpu.SemaphoreType.DMA((2,)),
        ],
        compiler_params=pltpu.CompilerParams(use_tc_tiling_on_sc=False),
    )(x_flat, E)


def kernel(x, E):
    flat = x.reshape(-1).astype(jnp.int32)
    out = _embed(flat, E)
    return out.reshape(x.shape[0], x.shape[1], E.shape[1])
